# baseline (speedup 1.0000x reference)
import functools

import jax
import jax.numpy as jnp
from jax import lax
from jax.experimental import pallas as pl
from jax.experimental.pallas import tpu as pltpu
from jax.experimental.pallas import tpu_sc as plsc

_N = 10000
_E = 320000
_D = 128
_NC = 2
_NS = 16
_NW = _NC * _NS
_EPW = _E // _NW
_CH = 80
_NCH = _EPW // _CH
_RB = 1000
_BN = 2000
_GRID = _N // _BN

_mesh = plsc.VectorSubcoreMesh(core_axis_name="c", subcore_axis_name="s")
_sc_params = pltpu.CompilerParams(use_tc_tiling_on_sc=False)


@functools.partial(
    pl.kernel,
    out_type=jax.ShapeDtypeStruct((_NC, _N, 16), jnp.float32),
    mesh=_mesh,
    compiler_params=_sc_params,
    scratch_types=[
        pltpu.VMEM((_NCH, _CH), jnp.int32),
        pltpu.VMEM((_CH, 16), jnp.float32),
        pltpu.VMEM((_RB, 16), jnp.float32),
        pltpu.VMEM_SHARED((_N, 16), jnp.float32),
    ],
)
def _sc_degree(dst_hbm, out_hbm, dstv, ones, zbuf, shared):
    c = lax.axis_index("c")
    s = lax.axis_index("s")
    wid = c * _NS + s
    pltpu.sync_copy(dst_hbm.at[wid], dstv)

    zero16 = jnp.zeros((16,), jnp.float32)
    ones16 = jnp.ones((16,), jnp.float32)

    def fill(i, _):
        zbuf[i, :] = zero16

        @pl.when(i < _CH)
        def _():
            ones[i, :] = ones16

        return 0

    lax.fori_loop(0, _RB, fill, 0)

    @pl.when(s < _N // _RB)
    def _():
        pltpu.sync_copy(zbuf, shared.at[pl.ds(s * _RB, _RB)])

    plsc.subcore_barrier()

    def hbody(j, _):
        pltpu.sync_copy(ones, shared.at[dstv.at[j]], add=True)
        return 0

    lax.fori_loop(0, _NCH, hbody, 0)
    plsc.subcore_barrier()

    @pl.when(s < _N // _RB)
    def _():
        pltpu.sync_copy(
            shared.at[pl.ds(s * _RB, _RB)], out_hbm.at[c, pl.ds(s * _RB, _RB)]
        )


@functools.partial(
    pl.kernel,
    out_type=jax.ShapeDtypeStruct((_NC, _N, _D), jnp.float32),
    mesh=_mesh,
    compiler_params=_sc_params,
    scratch_types=[
        pltpu.VMEM((_NCH, _CH), jnp.int32),
        pltpu.VMEM((_NCH, _CH), jnp.int32),
        pltpu.VMEM((_CH, _D), jnp.float32),
        pltpu.VMEM((_CH, _D), jnp.float32),
        pltpu.VMEM((_CH, _D), jnp.float32),
        pltpu.VMEM_SHARED((_N, _D), jnp.float32),
        pltpu.SemaphoreType.DMA,
        pltpu.SemaphoreType.DMA,
        pltpu.SemaphoreType.DMA,
        pltpu.SemaphoreType.DMA,
        pltpu.SemaphoreType.DMA,
        pltpu.SemaphoreType.DMA,
    ],
)
def _sc_scatter(y_hbm, src_hbm, dst_hbm, out_hbm, srcv, dstv,
                rows_a, rows_b, rows_c, acc, ga, gb, gc, sa, sb, sc):
    c = lax.axis_index("c")
    s = lax.axis_index("s")
    wid = c * _NS + s
    pltpu.sync_copy(src_hbm.at[wid], srcv)
    pltpu.sync_copy(dst_hbm.at[wid], dstv)

    @pl.when(s < _N // _RB)
    def _():
        pltpu.sync_copy(
            y_hbm.at[pl.ds(s * _RB, _RB)], acc.at[pl.ds(s * _RB, _RB)]
        )

    plsc.subcore_barrier()

    def gath(j, buf, gsem):
        pltpu.async_copy(y_hbm.at[srcv.at[j]], buf, gsem)

    def wait_g(buf, gsem):
        pltpu.make_async_copy(y_hbm.at[pl.ds(0, _CH)], buf, gsem).wait()

    def scat(j, buf, ssem):
        pltpu.async_copy(buf, acc.at[dstv.at[j]], ssem, add=True)

    def wait_s(buf, ssem):
        pltpu.make_async_copy(y_hbm.at[pl.ds(0, _CH)], buf, ssem).wait()

    gath(0, rows_a, ga)
    gath(1, rows_b, gb)

    def triple(k, _):
        j = 3 * k
        wait_g(rows_a, ga)
        scat(j, rows_a, sa)

        @pl.when(k > 0)
        def _():
            wait_s(rows_c, sc)

        gath(j + 2, rows_c, gc)
        wait_g(rows_b, gb)
        scat(j + 1, rows_b, sb)
        wait_s(rows_a, sa)
        gath(j + 3, rows_a, ga)
        wait_g(rows_c, gc)
        scat(j + 2, rows_c, sc)
        wait_s(rows_b, sb)
        gath(j + 4, rows_b, gb)
        return 0

    k_end = (_NCH - 2) // 3
    lax.fori_loop(0, k_end, triple, 0)
    wait_g(rows_a, ga)
    scat(_NCH - 2, rows_a, sa)
    wait_s(rows_c, sc)
    wait_g(rows_b, gb)
    scat(_NCH - 1, rows_b, sb)
    wait_s(rows_a, sa)
    wait_s(rows_b, sb)
    plsc.subcore_barrier()

    @pl.when(s < _N // _RB)
    def _():
        pltpu.sync_copy(
            acc.at[pl.ds(s * _RB, _RB)], out_hbm.at[c, pl.ds(s * _RB, _RB)]
        )


def _tc_mm1_body(x_ref, w_ref, xw_ref):
    xw_ref[...] = jnp.dot(
        x_ref[...], w_ref[...], preferred_element_type=jnp.float32
    )


def _tc_mm1(x, w1):
    return pl.pallas_call(
        _tc_mm1_body,
        grid=(_GRID,),
        in_specs=[
            pl.BlockSpec((_BN, _D), lambda i: (i, 0)),
            pl.BlockSpec((_D, _D), lambda i: (0, 0)),
        ],
        out_specs=pl.BlockSpec((_BN, _D), lambda i: (i, 0)),
        out_shape=jax.ShapeDtypeStruct((_N, _D), jnp.float32),
    )(x, w1)


def _tc_prep_body(dp_ref, xw_ref, y_ref, dinv_ref):
    d = dp_ref[0, :, 0:1] + dp_ref[1, :, 0:1] + 1.0
    dinv = lax.rsqrt(d)
    y_ref[...] = xw_ref[...] * dinv
    dinv_ref[...] = dinv


def _tc_prep(dp3, xw):
    return pl.pallas_call(
        _tc_prep_body,
        grid=(_GRID,),
        in_specs=[
            pl.BlockSpec((_NC, _BN, 16), lambda i: (0, i, 0)),
            pl.BlockSpec((_BN, _D), lambda i: (i, 0)),
        ],
        out_specs=[
            pl.BlockSpec((_BN, _D), lambda i: (i, 0)),
            pl.BlockSpec((_BN, 1), lambda i: (i, 0)),
        ],
        out_shape=[
            jax.ShapeDtypeStruct((_N, _D), jnp.float32),
            jax.ShapeDtypeStruct((_N, 1), jnp.float32),
        ],
    )(dp3, xw)


def _tc_mid_body(sp_ref, y_ref, dinv_ref, b_ref, w_ref, out_ref):
    m = sp_ref[0] + sp_ref[1] - y_ref[...]
    h = jnp.maximum(m * dinv_ref[...] + b_ref[...], 0.0)
    out_ref[...] = (
        jnp.dot(h, w_ref[...], preferred_element_type=jnp.float32)
        * dinv_ref[...]
    )


def _tc_mid(sp, y1, dinv, b1, w2):
    return pl.pallas_call(
        _tc_mid_body,
        grid=(_GRID,),
        in_specs=[
            pl.BlockSpec((_NC, _BN, _D), lambda i: (0, i, 0)),
            pl.BlockSpec((_BN, _D), lambda i: (i, 0)),
            pl.BlockSpec((_BN, 1), lambda i: (i, 0)),
            pl.BlockSpec((1, _D), lambda i: (0, 0)),
            pl.BlockSpec((_D, _D), lambda i: (0, 0)),
        ],
        out_specs=pl.BlockSpec((_BN, _D), lambda i: (i, 0)),
        out_shape=jax.ShapeDtypeStruct((_N, _D), jnp.float32),
    )(sp, y1, dinv, b1, w2)


def _tc_final_body(sp_ref, y_ref, dinv_ref, b_ref, out_ref):
    m = sp_ref[0] + sp_ref[1] - y_ref[...]
    h = m * dinv_ref[...]
    part = jnp.sum(h, axis=0, keepdims=True) * (1.0 / _N)

    @pl.when(pl.program_id(0) == 0)
    def _():
        out_ref[...] = b_ref[...]

    out_ref[...] += part


def _tc_final(sp, y2, dinv, b2):
    return pl.pallas_call(
        _tc_final_body,
        grid=(_GRID,),
        in_specs=[
            pl.BlockSpec((_NC, _BN, _D), lambda i: (0, i, 0)),
            pl.BlockSpec((_BN, _D), lambda i: (i, 0)),
            pl.BlockSpec((_BN, 1), lambda i: (i, 0)),
            pl.BlockSpec((1, _D), lambda i: (0, 0)),
        ],
        out_specs=pl.BlockSpec((1, _D), lambda i: (0, 0)),
        out_shape=jax.ShapeDtypeStruct((1, _D), jnp.float32),
    )(sp, y2, dinv, b2)


def kernel(x, edge_index, batch, W1, b1, W2, b2):
    src = edge_index[0]
    dst = edge_index[1]
    srcr = src.reshape(_NW, _NCH, _CH)
    dstr = dst.reshape(_NW, _NCH, _CH)

    xw = _tc_mm1(x, W1)
    dp = _sc_degree(dstr)
    y1, dinv = _tc_prep(dp, xw)
    s1 = _sc_scatter(y1, srcr, dstr)
    y2 = _tc_mid(s1, y1, dinv, b1.reshape(1, _D), W2)
    s2 = _sc_scatter(y2, srcr, dstr)
    return _tc_final(s2, y2, dinv, b2.reshape(1, _D))


# speedup vs baseline: 1.0007x; 1.0007x over previous
import functools

import jax
import jax.numpy as jnp
from jax import lax
from jax.experimental import pallas as pl
from jax.experimental.pallas import tpu as pltpu
from jax.experimental.pallas import tpu_sc as plsc

_N = 10000
_E = 320000
_D = 128
_NC = 2
_NS = 16
_NW = _NC * _NS
_EPW = _E // _NW
_CH = 80
_NCH = _EPW // _CH
_RB = 1000
_BN = 2000
_GRID = _N // _BN

_mesh = plsc.VectorSubcoreMesh(core_axis_name="c", subcore_axis_name="s")
_sc_params = pltpu.CompilerParams(use_tc_tiling_on_sc=False)


@functools.partial(
    pl.kernel,
    out_type=jax.ShapeDtypeStruct((_NC, _N, 16), jnp.float32),
    mesh=_mesh,
    compiler_params=_sc_params,
    scratch_types=[
        pltpu.VMEM((_NCH, _CH), jnp.int32),
        pltpu.VMEM((_CH, 16), jnp.float32),
        pltpu.VMEM((_RB, 16), jnp.float32),
        pltpu.VMEM_SHARED((_N, 16), jnp.float32),
    ],
)
def _sc_degree(dst_hbm, out_hbm, dstv, ones, zbuf, shared):
    c = lax.axis_index("c")
    s = lax.axis_index("s")
    wid = c * _NS + s
    pltpu.sync_copy(dst_hbm.at[wid], dstv)

    zero16 = jnp.zeros((16,), jnp.float32)
    ones16 = jnp.ones((16,), jnp.float32)

    def fill(i, _):
        zbuf[i, :] = zero16

        @pl.when(i < _CH)
        def _():
            ones[i, :] = ones16

        return 0

    lax.fori_loop(0, _RB, fill, 0)

    @pl.when(s < _N // _RB)
    def _():
        pltpu.sync_copy(zbuf, shared.at[pl.ds(s * _RB, _RB)])

    plsc.subcore_barrier()

    def hbody(j, _):
        pltpu.sync_copy(ones, shared.at[dstv.at[j]], add=True)
        return 0

    lax.fori_loop(0, _NCH, hbody, 0)
    plsc.subcore_barrier()

    @pl.when(s < _N // _RB)
    def _():
        pltpu.sync_copy(
            shared.at[pl.ds(s * _RB, _RB)], out_hbm.at[c, pl.ds(s * _RB, _RB)]
        )


@functools.partial(
    pl.kernel,
    out_type=jax.ShapeDtypeStruct((_NC, _N, _D), jnp.float32),
    mesh=_mesh,
    compiler_params=_sc_params,
    scratch_types=[
        pltpu.VMEM((_NCH, _CH), jnp.int32),
        pltpu.VMEM((_NCH, _CH), jnp.int32),
        pltpu.VMEM((_CH, _D), jnp.float32),
        pltpu.VMEM((_CH, _D), jnp.float32),
        pltpu.VMEM((_CH, _D), jnp.float32),
        pltpu.VMEM_SHARED((_N, _D), jnp.float32),
        pltpu.SemaphoreType.DMA,
        pltpu.SemaphoreType.DMA,
        pltpu.SemaphoreType.DMA,
        pltpu.SemaphoreType.DMA,
        pltpu.SemaphoreType.DMA,
        pltpu.SemaphoreType.DMA,
    ],
)
def _sc_scatter(y_hbm, src_hbm, dst_hbm, out_hbm, srcv, dstv,
                rows_a, rows_b, rows_c, acc, ga, gb, gc, sa, sb, sc):
    c = lax.axis_index("c")
    s = lax.axis_index("s")
    wid = c * _NS + s
    pltpu.sync_copy(src_hbm.at[wid], srcv)
    pltpu.sync_copy(dst_hbm.at[wid], dstv)

    @pl.when(s < _N // _RB)
    def _():
        pltpu.sync_copy(
            y_hbm.at[pl.ds(s * _RB, _RB)], acc.at[pl.ds(s * _RB, _RB)]
        )

    plsc.subcore_barrier()

    def gath(j, buf, gsem):
        pltpu.async_copy(y_hbm.at[srcv.at[j]], buf, gsem)

    def wait_g(buf, gsem):
        pltpu.make_async_copy(y_hbm.at[pl.ds(0, _CH)], buf, gsem).wait()

    def scat(j, buf, ssem):
        pltpu.async_copy(buf, acc.at[dstv.at[j]], ssem, add=True)

    def wait_s(buf, ssem):
        pltpu.make_async_copy(y_hbm.at[pl.ds(0, _CH)], buf, ssem).wait()

    gath(0, rows_a, ga)
    gath(1, rows_b, gb)

    def triple(k, _):
        j = 3 * k
        wait_g(rows_a, ga)
        scat(j, rows_a, sa)

        @pl.when(k > 0)
        def _():
            wait_s(rows_c, sc)

        gath(j + 2, rows_c, gc)
        wait_g(rows_b, gb)
        scat(j + 1, rows_b, sb)
        wait_s(rows_a, sa)
        gath(j + 3, rows_a, ga)
        wait_g(rows_c, gc)
        scat(j + 2, rows_c, sc)
        wait_s(rows_b, sb)
        gath(j + 4, rows_b, gb)
        return 0

    k_end = (_NCH - 2) // 3
    lax.fori_loop(0, k_end, triple, 0)
    wait_g(rows_a, ga)
    scat(_NCH - 2, rows_a, sa)
    wait_s(rows_c, sc)
    wait_g(rows_b, gb)
    scat(_NCH - 1, rows_b, sb)
    wait_s(rows_a, sa)
    wait_s(rows_b, sb)
    plsc.subcore_barrier()

    @pl.when(s < _N // _RB)
    def _():
        pltpu.sync_copy(
            acc.at[pl.ds(s * _RB, _RB)], out_hbm.at[c, pl.ds(s * _RB, _RB)]
        )


def _tc_prep_body(dp_ref, x_ref, w_ref, y_ref, dinv_ref):
    d = dp_ref[0, :, 0:1] + dp_ref[1, :, 0:1] + 1.0
    dinv = lax.rsqrt(d)
    xw = jnp.dot(x_ref[...], w_ref[...], preferred_element_type=jnp.float32)
    y_ref[...] = xw * dinv
    dinv_ref[...] = dinv


def _tc_prep(dp3, x, w1):
    return pl.pallas_call(
        _tc_prep_body,
        grid=(_GRID,),
        in_specs=[
            pl.BlockSpec((_NC, _BN, 16), lambda i: (0, i, 0)),
            pl.BlockSpec((_BN, _D), lambda i: (i, 0)),
            pl.BlockSpec((_D, _D), lambda i: (0, 0)),
        ],
        out_specs=[
            pl.BlockSpec((_BN, _D), lambda i: (i, 0)),
            pl.BlockSpec((_BN, 1), lambda i: (i, 0)),
        ],
        out_shape=[
            jax.ShapeDtypeStruct((_N, _D), jnp.float32),
            jax.ShapeDtypeStruct((_N, 1), jnp.float32),
        ],
    )(dp3, x, w1)


def _tc_mid_body(sp_ref, y_ref, dinv_ref, b_ref, w_ref, out_ref):
    m = sp_ref[0] + sp_ref[1] - y_ref[...]
    h = jnp.maximum(m * dinv_ref[...] + b_ref[...], 0.0)
    out_ref[...] = (
        jnp.dot(h, w_ref[...], preferred_element_type=jnp.float32)
        * dinv_ref[...]
    )


def _tc_mid(sp, y1, dinv, b1, w2):
    return pl.pallas_call(
        _tc_mid_body,
        grid=(_GRID,),
        in_specs=[
            pl.BlockSpec((_NC, _BN, _D), lambda i: (0, i, 0)),
            pl.BlockSpec((_BN, _D), lambda i: (i, 0)),
            pl.BlockSpec((_BN, 1), lambda i: (i, 0)),
            pl.BlockSpec((1, _D), lambda i: (0, 0)),
            pl.BlockSpec((_D, _D), lambda i: (0, 0)),
        ],
        out_specs=pl.BlockSpec((_BN, _D), lambda i: (i, 0)),
        out_shape=jax.ShapeDtypeStruct((_N, _D), jnp.float32),
    )(sp, y1, dinv, b1, w2)


def _tc_final_body(sp_ref, y_ref, dinv_ref, b_ref, out_ref):
    m = sp_ref[0] + sp_ref[1] - y_ref[...]
    h = m * dinv_ref[...]
    part = jnp.sum(h, axis=0, keepdims=True) * (1.0 / _N)

    @pl.when(pl.program_id(0) == 0)
    def _():
        out_ref[...] = b_ref[...]

    out_ref[...] += part


def _tc_final(sp, y2, dinv, b2):
    return pl.pallas_call(
        _tc_final_body,
        grid=(_GRID,),
        in_specs=[
            pl.BlockSpec((_NC, _BN, _D), lambda i: (0, i, 0)),
            pl.BlockSpec((_BN, _D), lambda i: (i, 0)),
            pl.BlockSpec((_BN, 1), lambda i: (i, 0)),
            pl.BlockSpec((1, _D), lambda i: (0, 0)),
        ],
        out_specs=pl.BlockSpec((1, _D), lambda i: (0, 0)),
        out_shape=jax.ShapeDtypeStruct((1, _D), jnp.float32),
    )(sp, y2, dinv, b2)


def kernel(x, edge_index, batch, W1, b1, W2, b2):
    src = edge_index[0]
    dst = edge_index[1]
    srcr = src.reshape(_NW, _NCH, _CH)
    dstr = dst.reshape(_NW, _NCH, _CH)

    dp = _sc_degree(dstr)
    y1, dinv = _tc_prep(dp, x, W1)
    s1 = _sc_scatter(y1, srcr, dstr)
    y2 = _tc_mid(s1, y1, dinv, b1.reshape(1, _D), W2)
    s2 = _sc_scatter(y2, srcr, dstr)
    return _tc_final(s2, y2, dinv, b2.reshape(1, _D))


# speedup vs baseline: 1.0254x; 1.0247x over previous
import functools

import jax
import jax.numpy as jnp
from jax import lax
from jax.experimental import pallas as pl
from jax.experimental.pallas import tpu as pltpu
from jax.experimental.pallas import tpu_sc as plsc

_N = 10000
_E = 320000
_D = 128
_NC = 2
_NS = 16
_NW = _NC * _NS
_EPW = _E // _NW
_CH = 80
_NCH = _EPW // _CH
_RB = 1000
_BN = 2000
_GRID = _N // _BN

_mesh = plsc.VectorSubcoreMesh(core_axis_name="c", subcore_axis_name="s")
_sc_params = pltpu.CompilerParams(use_tc_tiling_on_sc=False)


@functools.partial(
    pl.kernel,
    out_type=jax.ShapeDtypeStruct((_NC, _N, 16), jnp.float32),
    mesh=_mesh,
    compiler_params=_sc_params,
    scratch_types=[
        pltpu.VMEM((_NCH, _CH), jnp.int32),
        pltpu.VMEM((_CH, 16), jnp.float32),
        pltpu.VMEM((_RB, 16), jnp.float32),
        pltpu.VMEM_SHARED((_N, 16), jnp.float32),
        pltpu.SemaphoreType.DMA,
    ],
)
def _sc_degree(dst_hbm, out_hbm, dstv, ones, zbuf, shared, sem):
    c = lax.axis_index("c")
    s = lax.axis_index("s")
    wid = c * _NS + s
    pltpu.sync_copy(dst_hbm.at[wid], dstv)

    zero16 = jnp.zeros((16,), jnp.float32)
    ones16 = jnp.ones((16,), jnp.float32)

    def fill_ones(i, _):
        ones[i, :] = ones16
        return 0

    lax.fori_loop(0, _CH, fill_ones, 0)

    @pl.when(s < _N // _RB)
    def _():
        def fill_zero(i, _):
            zbuf[i, :] = zero16
            return 0

        lax.fori_loop(0, _RB, fill_zero, 0)
        pltpu.sync_copy(zbuf, shared.at[pl.ds(s * _RB, _RB)])

    plsc.subcore_barrier()

    def drain():
        pltpu.make_async_copy(out_hbm.at[0, pl.ds(0, _CH)], ones, sem).wait()

    def hbody(j, _):
        pltpu.async_copy(ones, shared.at[dstv.at[j]], sem, add=True)

        @pl.when(j >= 7)
        def _():
            drain()

        return 0

    lax.fori_loop(0, _NCH, hbody, 0)

    def tail(j, _):
        drain()
        return 0

    lax.fori_loop(0, 7, tail, 0)
    plsc.subcore_barrier()

    @pl.when(s < _N // _RB)
    def _():
        pltpu.sync_copy(
            shared.at[pl.ds(s * _RB, _RB)], out_hbm.at[c, pl.ds(s * _RB, _RB)]
        )


@functools.partial(
    pl.kernel,
    out_type=jax.ShapeDtypeStruct((_NC, _N, _D), jnp.float32),
    mesh=_mesh,
    compiler_params=_sc_params,
    scratch_types=[
        pltpu.VMEM((_NCH, _CH), jnp.int32),
        pltpu.VMEM((_NCH, _CH), jnp.int32),
        pltpu.VMEM((_CH, _D), jnp.float32),
        pltpu.VMEM((_CH, _D), jnp.float32),
        pltpu.VMEM((_CH, _D), jnp.float32),
        pltpu.VMEM_SHARED((_N, _D), jnp.float32),
        pltpu.SemaphoreType.DMA,
        pltpu.SemaphoreType.DMA,
        pltpu.SemaphoreType.DMA,
        pltpu.SemaphoreType.DMA,
        pltpu.SemaphoreType.DMA,
        pltpu.SemaphoreType.DMA,
    ],
)
def _sc_scatter(y_hbm, src_hbm, dst_hbm, out_hbm, srcv, dstv,
                rows_a, rows_b, rows_c, acc, ga, gb, gc, sa, sb, sc):
    c = lax.axis_index("c")
    s = lax.axis_index("s")
    wid = c * _NS + s
    pltpu.sync_copy(src_hbm.at[wid], srcv)
    pltpu.sync_copy(dst_hbm.at[wid], dstv)

    @pl.when(s < _N // _RB)
    def _():
        pltpu.sync_copy(
            y_hbm.at[pl.ds(s * _RB, _RB)], acc.at[pl.ds(s * _RB, _RB)]
        )

    plsc.subcore_barrier()

    def gath(j, buf, gsem):
        pltpu.async_copy(y_hbm.at[srcv.at[j]], buf, gsem)

    def wait_g(buf, gsem):
        pltpu.make_async_copy(y_hbm.at[pl.ds(0, _CH)], buf, gsem).wait()

    def scat(j, buf, ssem):
        pltpu.async_copy(buf, acc.at[dstv.at[j]], ssem, add=True)

    def wait_s(buf, ssem):
        pltpu.make_async_copy(y_hbm.at[pl.ds(0, _CH)], buf, ssem).wait()

    gath(0, rows_a, ga)
    gath(1, rows_b, gb)

    def triple(k, _):
        j = 3 * k
        wait_g(rows_a, ga)
        scat(j, rows_a, sa)

        @pl.when(k > 0)
        def _():
            wait_s(rows_c, sc)

        gath(j + 2, rows_c, gc)
        wait_g(rows_b, gb)
        scat(j + 1, rows_b, sb)
        wait_s(rows_a, sa)
        gath(j + 3, rows_a, ga)
        wait_g(rows_c, gc)
        scat(j + 2, rows_c, sc)
        wait_s(rows_b, sb)
        gath(j + 4, rows_b, gb)
        return 0

    k_end = (_NCH - 2) // 3
    lax.fori_loop(0, k_end, triple, 0)
    wait_g(rows_a, ga)
    scat(_NCH - 2, rows_a, sa)
    wait_s(rows_c, sc)
    wait_g(rows_b, gb)
    scat(_NCH - 1, rows_b, sb)
    wait_s(rows_a, sa)
    wait_s(rows_b, sb)
    plsc.subcore_barrier()

    @pl.when(s < _N // _RB)
    def _():
        pltpu.sync_copy(
            acc.at[pl.ds(s * _RB, _RB)], out_hbm.at[c, pl.ds(s * _RB, _RB)]
        )


def _tc_prep_body(dp_ref, x_ref, w_ref, y_ref, dinv_ref):
    d = dp_ref[0, :, 0:1] + dp_ref[1, :, 0:1] + 1.0
    dinv = lax.rsqrt(d)
    xw = jnp.dot(x_ref[...], w_ref[...], preferred_element_type=jnp.float32)
    y_ref[...] = xw * dinv
    dinv_ref[...] = dinv


def _tc_prep(dp3, x, w1):
    return pl.pallas_call(
        _tc_prep_body,
        grid=(_GRID,),
        in_specs=[
            pl.BlockSpec((_NC, _BN, 16), lambda i: (0, i, 0)),
            pl.BlockSpec((_BN, _D), lambda i: (i, 0)),
            pl.BlockSpec((_D, _D), lambda i: (0, 0)),
        ],
        out_specs=[
            pl.BlockSpec((_BN, _D), lambda i: (i, 0)),
            pl.BlockSpec((_BN, 1), lambda i: (i, 0)),
        ],
        out_shape=[
            jax.ShapeDtypeStruct((_N, _D), jnp.float32),
            jax.ShapeDtypeStruct((_N, 1), jnp.float32),
        ],
    )(dp3, x, w1)


def _tc_mid_body(sp_ref, y_ref, dinv_ref, b_ref, w_ref, out_ref):
    m = sp_ref[0] + sp_ref[1] - y_ref[...]
    h = jnp.maximum(m * dinv_ref[...] + b_ref[...], 0.0)
    out_ref[...] = (
        jnp.dot(h, w_ref[...], preferred_element_type=jnp.float32)
        * dinv_ref[...]
    )


def _tc_mid(sp, y1, dinv, b1, w2):
    return pl.pallas_call(
        _tc_mid_body,
        grid=(_GRID,),
        in_specs=[
            pl.BlockSpec((_NC, _BN, _D), lambda i: (0, i, 0)),
            pl.BlockSpec((_BN, _D), lambda i: (i, 0)),
            pl.BlockSpec((_BN, 1), lambda i: (i, 0)),
            pl.BlockSpec((1, _D), lambda i: (0, 0)),
            pl.BlockSpec((_D, _D), lambda i: (0, 0)),
        ],
        out_specs=pl.BlockSpec((_BN, _D), lambda i: (i, 0)),
        out_shape=jax.ShapeDtypeStruct((_N, _D), jnp.float32),
    )(sp, y1, dinv, b1, w2)


def _tc_final_body(sp_ref, y_ref, dinv_ref, b_ref, out_ref):
    m = sp_ref[0] + sp_ref[1] - y_ref[...]
    h = m * dinv_ref[...]
    part = jnp.sum(h, axis=0, keepdims=True) * (1.0 / _N)

    @pl.when(pl.program_id(0) == 0)
    def _():
        out_ref[...] = b_ref[...]

    out_ref[...] += part


def _tc_final(sp, y2, dinv, b2):
    return pl.pallas_call(
        _tc_final_body,
        grid=(_GRID,),
        in_specs=[
            pl.BlockSpec((_NC, _BN, _D), lambda i: (0, i, 0)),
            pl.BlockSpec((_BN, _D), lambda i: (i, 0)),
            pl.BlockSpec((_BN, 1), lambda i: (i, 0)),
            pl.BlockSpec((1, _D), lambda i: (0, 0)),
        ],
        out_specs=pl.BlockSpec((1, _D), lambda i: (0, 0)),
        out_shape=jax.ShapeDtypeStruct((1, _D), jnp.float32),
    )(sp, y2, dinv, b2)


def kernel(x, edge_index, batch, W1, b1, W2, b2):
    src = edge_index[0]
    dst = edge_index[1]
    srcr = src.reshape(_NW, _NCH, _CH)
    dstr = dst.reshape(_NW, _NCH, _CH)

    dp = _sc_degree(dstr)
    y1, dinv = _tc_prep(dp, x, W1)
    s1 = _sc_scatter(y1, srcr, dstr)
    y2 = _tc_mid(s1, y1, dinv, b1.reshape(1, _D), W2)
    s2 = _sc_scatter(y2, srcr, dstr)
    return _tc_final(s2, y2, dinv, b2.reshape(1, _D))


# speedup vs baseline: 1.0402x; 1.0144x over previous
import functools

import jax
import jax.numpy as jnp
from jax import lax
from jax.experimental import pallas as pl
from jax.experimental.pallas import tpu as pltpu
from jax.experimental.pallas import tpu_sc as plsc

_N = 10000
_E = 320000
_D = 128
_NC = 2
_NS = 16
_NW = _NC * _NS
_EPW = _E // _NW
_CH = 80
_NCH = _EPW // _CH
_RB = 1000
_BN = 2000
_GRID = _N // _BN

_mesh = plsc.VectorSubcoreMesh(core_axis_name="c", subcore_axis_name="s")
_sc_params = pltpu.CompilerParams(use_tc_tiling_on_sc=False)


@functools.partial(
    pl.kernel,
    out_type=jax.ShapeDtypeStruct((_NC, _N, 16), jnp.float32),
    mesh=_mesh,
    compiler_params=_sc_params,
    scratch_types=[
        pltpu.VMEM((_NCH, _CH), jnp.int32),
        pltpu.VMEM((_CH, 16), jnp.float32),
        pltpu.VMEM((_RB, 16), jnp.float32),
        pltpu.VMEM_SHARED((_N, 16), jnp.float32),
        pltpu.SemaphoreType.DMA,
    ],
)
def _sc_degree(dst_hbm, out_hbm, dstv, ones, zbuf, shared, sem):
    c = lax.axis_index("c")
    s = lax.axis_index("s")
    wid = c * _NS + s
    idx_cp = pltpu.async_copy(dst_hbm.at[wid], dstv, sem)

    zero16 = jnp.zeros((16,), jnp.float32)
    ones16 = jnp.ones((16,), jnp.float32)

    def fill_ones(i, _):
        ones[i, :] = ones16
        return 0

    lax.fori_loop(0, _CH, fill_ones, 0)
    idx_cp.wait()

    @pl.when(s < _N // _RB)
    def _():
        def fill_zero(i, _):
            zbuf[i, :] = zero16
            return 0

        lax.fori_loop(0, _RB, fill_zero, 0)
        pltpu.sync_copy(zbuf, shared.at[pl.ds(s * _RB, _RB)])

    plsc.subcore_barrier()

    def drain():
        pltpu.make_async_copy(out_hbm.at[0, pl.ds(0, _CH)], ones, sem).wait()

    def hbody(j, _):
        pltpu.async_copy(ones, shared.at[dstv.at[j]], sem, add=True)

        @pl.when(j >= 7)
        def _():
            drain()

        return 0

    lax.fori_loop(0, _NCH, hbody, 0)

    def tail(j, _):
        drain()
        return 0

    lax.fori_loop(0, 7, tail, 0)
    plsc.subcore_barrier()

    @pl.when(s < _N // _RB)
    def _():
        pltpu.sync_copy(
            shared.at[pl.ds(s * _RB, _RB)], out_hbm.at[c, pl.ds(s * _RB, _RB)]
        )


@functools.partial(
    pl.kernel,
    out_type=jax.ShapeDtypeStruct((_NC, _N, _D), jnp.float32),
    mesh=_mesh,
    compiler_params=_sc_params,
    scratch_types=[
        pltpu.VMEM((_NCH, _CH), jnp.int32),
        pltpu.VMEM((_NCH, _CH), jnp.int32),
        pltpu.VMEM((_CH, _D), jnp.float32),
        pltpu.VMEM((_CH, _D), jnp.float32),
        pltpu.VMEM((_CH, _D), jnp.float32),
        pltpu.VMEM_SHARED((_N, _D), jnp.float32),
        pltpu.SemaphoreType.DMA,
        pltpu.SemaphoreType.DMA,
        pltpu.SemaphoreType.DMA,
        pltpu.SemaphoreType.DMA,
        pltpu.SemaphoreType.DMA,
        pltpu.SemaphoreType.DMA,
    ],
)
def _sc_scatter(y_hbm, src_hbm, dst_hbm, out_hbm, srcv, dstv,
                rows_a, rows_b, rows_c, acc, ga, gb, gc, sa, sb, sc):
    c = lax.axis_index("c")
    s = lax.axis_index("s")
    wid = c * _NS + s
    src_cp = pltpu.async_copy(src_hbm.at[wid], srcv, ga)
    dst_cp = pltpu.async_copy(dst_hbm.at[wid], dstv, gb)

    @pl.when(s < _N // _RB)
    def _():
        pltpu.sync_copy(
            y_hbm.at[pl.ds(s * _RB, _RB)], acc.at[pl.ds(s * _RB, _RB)]
        )

    src_cp.wait()
    dst_cp.wait()
    plsc.subcore_barrier()

    def gath(j, buf, gsem):
        pltpu.async_copy(y_hbm.at[srcv.at[j]], buf, gsem)

    def wait_g(buf, gsem):
        pltpu.make_async_copy(y_hbm.at[pl.ds(0, _CH)], buf, gsem).wait()

    def scat(j, buf, ssem):
        pltpu.async_copy(buf, acc.at[dstv.at[j]], ssem, add=True)

    def wait_s(buf, ssem):
        pltpu.make_async_copy(y_hbm.at[pl.ds(0, _CH)], buf, ssem).wait()

    gath(0, rows_a, ga)
    gath(1, rows_b, gb)

    def triple(k, _):
        j = 3 * k
        wait_g(rows_a, ga)
        scat(j, rows_a, sa)

        @pl.when(k > 0)
        def _():
            wait_s(rows_c, sc)

        gath(j + 2, rows_c, gc)
        wait_g(rows_b, gb)
        scat(j + 1, rows_b, sb)
        wait_s(rows_a, sa)
        gath(j + 3, rows_a, ga)
        wait_g(rows_c, gc)
        scat(j + 2, rows_c, sc)
        wait_s(rows_b, sb)
        gath(j + 4, rows_b, gb)
        return 0

    k_end = (_NCH - 2) // 3
    lax.fori_loop(0, k_end, triple, 0)
    wait_g(rows_a, ga)
    scat(_NCH - 2, rows_a, sa)
    wait_s(rows_c, sc)
    wait_g(rows_b, gb)
    scat(_NCH - 1, rows_b, sb)
    wait_s(rows_a, sa)
    wait_s(rows_b, sb)
    plsc.subcore_barrier()

    @pl.when(s < _N // _RB)
    def _():
        pltpu.sync_copy(
            acc.at[pl.ds(s * _RB, _RB)], out_hbm.at[c, pl.ds(s * _RB, _RB)]
        )


def _tc_prep_body(dp_ref, x_ref, w_ref, y_ref, dinv_ref):
    d = dp_ref[0, :, 0:1] + dp_ref[1, :, 0:1] + 1.0
    dinv = lax.rsqrt(d)
    xw = jnp.dot(x_ref[...], w_ref[...], preferred_element_type=jnp.float32)
    y_ref[...] = xw * dinv
    dinv_ref[...] = dinv


def _tc_prep(dp3, x, w1):
    return pl.pallas_call(
        _tc_prep_body,
        grid=(_GRID,),
        in_specs=[
            pl.BlockSpec((_NC, _BN, 16), lambda i: (0, i, 0)),
            pl.BlockSpec((_BN, _D), lambda i: (i, 0)),
            pl.BlockSpec((_D, _D), lambda i: (0, 0)),
        ],
        out_specs=[
            pl.BlockSpec((_BN, _D), lambda i: (i, 0)),
            pl.BlockSpec((_BN, 1), lambda i: (i, 0)),
        ],
        out_shape=[
            jax.ShapeDtypeStruct((_N, _D), jnp.float32),
            jax.ShapeDtypeStruct((_N, 1), jnp.float32),
        ],
    )(dp3, x, w1)


def _tc_mid_body(sp_ref, y_ref, dinv_ref, b_ref, w_ref, out_ref):
    m = sp_ref[0] + sp_ref[1] - y_ref[...]
    h = jnp.maximum(m * dinv_ref[...] + b_ref[...], 0.0)
    out_ref[...] = (
        jnp.dot(h, w_ref[...], preferred_element_type=jnp.float32)
        * dinv_ref[...]
    )


def _tc_mid(sp, y1, dinv, b1, w2):
    return pl.pallas_call(
        _tc_mid_body,
        grid=(_GRID,),
        in_specs=[
            pl.BlockSpec((_NC, _BN, _D), lambda i: (0, i, 0)),
            pl.BlockSpec((_BN, _D), lambda i: (i, 0)),
            pl.BlockSpec((_BN, 1), lambda i: (i, 0)),
            pl.BlockSpec((1, _D), lambda i: (0, 0)),
            pl.BlockSpec((_D, _D), lambda i: (0, 0)),
        ],
        out_specs=pl.BlockSpec((_BN, _D), lambda i: (i, 0)),
        out_shape=jax.ShapeDtypeStruct((_N, _D), jnp.float32),
    )(sp, y1, dinv, b1, w2)


def _tc_final_body(sp_ref, y_ref, dinv_ref, b_ref, out_ref):
    m = sp_ref[0] + sp_ref[1] - y_ref[...]
    h = m * dinv_ref[...]
    part = jnp.sum(h, axis=0, keepdims=True) * (1.0 / _N)

    @pl.when(pl.program_id(0) == 0)
    def _():
        out_ref[...] = b_ref[...]

    out_ref[...] += part


def _tc_final(sp, y2, dinv, b2):
    return pl.pallas_call(
        _tc_final_body,
        grid=(_GRID,),
        in_specs=[
            pl.BlockSpec((_NC, _BN, _D), lambda i: (0, i, 0)),
            pl.BlockSpec((_BN, _D), lambda i: (i, 0)),
            pl.BlockSpec((_BN, 1), lambda i: (i, 0)),
            pl.BlockSpec((1, _D), lambda i: (0, 0)),
        ],
        out_specs=pl.BlockSpec((1, _D), lambda i: (0, 0)),
        out_shape=jax.ShapeDtypeStruct((1, _D), jnp.float32),
    )(sp, y2, dinv, b2)


def kernel(x, edge_index, batch, W1, b1, W2, b2):
    src = edge_index[0]
    dst = edge_index[1]
    srcr = src.reshape(_NW, _NCH, _CH)
    dstr = dst.reshape(_NW, _NCH, _CH)

    dp = _sc_degree(dstr)
    y1, dinv = _tc_prep(dp, x, W1)
    s1 = _sc_scatter(y1, srcr, dstr)
    y2 = _tc_mid(s1, y1, dinv, b1.reshape(1, _D), W2)
    s2 = _sc_scatter(y2, srcr, dstr)
    return _tc_final(s2, y2, dinv, b2.reshape(1, _D))
